# baseline (speedup 1.0000x reference)
import functools

import jax
import jax.numpy as jnp
from jax import lax
from jax.experimental import pallas as pl
from jax.experimental.pallas import tpu as pltpu
from jax.experimental.pallas import tpu_sc as plsc

D = 1024
NC, NS = 2, 16
NW = NC * NS
C = 32
NBUF = 3


@functools.partial(jax.jit, static_argnames=("B",))
def _gather(idx, table, B):
    n_per_w = B // NW
    n_chunks = n_per_w // C
    mesh = plsc.VectorSubcoreMesh(core_axis_name="c", subcore_axis_name="s")

    @functools.partial(
        pl.kernel,
        out_type=jax.ShapeDtypeStruct((B, D), jnp.float32),
        mesh=mesh,
        scratch_types=[
            pltpu.VMEM((n_chunks, C), jnp.int32),
            pltpu.VMEM((NBUF, C, D), jnp.float32),
            [pltpu.SemaphoreType.DMA] * NBUF,
            [pltpu.SemaphoreType.DMA] * NBUF,
        ],
    )
    def k(idx_hbm, table_hbm, out_hbm, idx_v, buf, gsems, ssems):
        wid = lax.axis_index("s") * NC + lax.axis_index("c")
        base = wid * n_per_w
        pltpu.sync_copy(idx_hbm.at[wid], idx_v)

        def start_gather(i, b):
            pltpu.async_copy(table_hbm.at[idx_v.at[i]], buf.at[b], gsems[b])

        def one_chunk(i, b):
            pltpu.make_async_copy(
                table_hbm.at[idx_v.at[i]], buf.at[b], gsems[b]
            ).wait()
            pltpu.async_copy(
                buf.at[b], out_hbm.at[pl.ds(base + i * C, C)], ssems[b]
            )
            bp = (b + NBUF - 1) % NBUF

            @pl.when(i >= 1)
            def _():
                pltpu.make_async_copy(
                    buf.at[bp],
                    out_hbm.at[pl.ds(base + (i - 1) * C, C)],
                    ssems[bp],
                ).wait()

            @pl.when(i + 2 < n_chunks)
            def _():
                start_gather(i + 2, bp)

        start_gather(0, 0)
        start_gather(1, 1)

        n_main = (n_chunks // NBUF) * NBUF

        @pl.loop(0, n_main, step=NBUF)
        def _(j):
            for b in range(NBUF):
                one_chunk(j + b, b)

        for i in range(n_main, n_chunks):
            one_chunk(i, i % NBUF)

        blast = (n_chunks - 1) % NBUF
        pltpu.make_async_copy(
            buf.at[blast],
            out_hbm.at[pl.ds(base + (n_chunks - 1) * C, C)],
            ssems[blast],
        ).wait()

    return k(idx, table)


def kernel(inputs, embedding):
    B = inputs.size
    n_per_w = B // NW
    idx = inputs.reshape(NW, n_per_w // C, C).astype(jnp.int32)
    out = _gather(idx, embedding, B)
    return out.reshape(*inputs.shape, D)
